# baseline (speedup 1.0000x reference)
import jax
import jax.numpy as jnp
from jax.experimental import pallas as pl
from jax.experimental.pallas import tpu as pltpu


def _body(xt_ref, hand_hbm, act_hbm, out_ref, hand_v, act_v, s0, s1):
    c0 = pltpu.make_async_copy(hand_hbm, hand_v, s0)
    c1 = pltpu.make_async_copy(act_hbm, act_v, s1)
    c0.start()
    c1.start()
    t = jnp.transpose(xt_ref[...])
    hi = t[:, 0:1].astype(jnp.int32)
    ai = t[:, 1:2].astype(jnp.int32)
    c0.wait()
    c1.wait()
    h = jnp.zeros((10, 255), jnp.float32)
    for v in range(5):
        h = jnp.where(hi == v, hand_v[v, :][None, :], h)
    a = jnp.zeros((10, 256), jnp.float32)
    for v in range(6):
        a = jnp.where(ai == v, act_v[v, :][None, :], a)
    out_ref[...] = jnp.concatenate([h, a, t[:, 2:3]], axis=1)


def kernel(x, hand_table, action_table):
    xt = x[0].T
    return pl.pallas_call(
        _body,
        in_specs=[
            pl.BlockSpec(memory_space=pltpu.MemorySpace.VMEM),
            pl.BlockSpec(memory_space=pltpu.MemorySpace.HBM),
            pl.BlockSpec(memory_space=pltpu.MemorySpace.HBM),
        ],
        out_shape=jax.ShapeDtypeStruct((10, 512), jnp.float32),
        scratch_shapes=[
            pltpu.VMEM((5, 255), jnp.float32),
            pltpu.VMEM((6, 256), jnp.float32),
            pltpu.SemaphoreType.DMA,
            pltpu.SemaphoreType.DMA,
        ],
        compiler_params=pltpu.CompilerParams(
            allow_input_fusion=[True, False, False]
        ),
    )(xt, hand_table, action_table)
